# baseline (speedup 1.0000x reference)
import functools
import math

import jax
import jax.numpy as jnp
from jax import lax
from jax.experimental import pallas as pl
from jax.experimental.pallas import tpu as pltpu
from jax.experimental.pallas import tpu_sc as plsc

D_MODEL = 128
SCALE = math.sqrt(float(D_MODEL))

_NC = 2
_NS = 16
_NW = _NC * _NS

N_SPLIT = 4
SP = 56
NBUF = 8
LEAD = 6
BLK = 32


def _make_sc_gather(rows, S, D):
    assert rows % _NW == 0
    rpw = rows // _NW

    mesh = plsc.VectorSubcoreMesh(core_axis_name="c", subcore_axis_name="s")

    @functools.partial(
        pl.kernel,
        mesh=mesh,
        out_type=jax.ShapeDtypeStruct((rows, SP, D), jnp.float32),
        scratch_types=[
            pltpu.VMEM((rpw, S), jnp.int32),
            *([pltpu.VMEM((SP, D), jnp.float32)] * NBUF),
            *([pltpu.SemaphoreType.DMA] * NBUF),
            *([pltpu.SemaphoreType.DMA] * NBUF),
        ],
    )
    def sc_gather(table_hbm, idx_hbm, y_hbm, idx_v, *bufs):
        buf = bufs[:NBUF]
        gsem = bufs[NBUF:2 * NBUF]
        ssem = bufs[2 * NBUF:3 * NBUF]

        wid = lax.axis_index("s") * _NC + lax.axis_index("c")
        pltpu.sync_copy(idx_hbm.at[wid], idx_v)
        row0 = wid * rpw

        def issue_gather(r):
            b = r % NBUF
            pltpu.make_async_copy(
                table_hbm.at[idx_v.at[r]],
                buf[b].at[pl.ds(0, S)], gsem[b]).start()

        def wait_gather(r):
            b = r % NBUF
            pltpu.make_async_copy(
                table_hbm.at[idx_v.at[0]],
                buf[b].at[pl.ds(0, S)], gsem[b]).wait()

        def issue_scatter(r):
            b = r % NBUF
            pltpu.make_async_copy(buf[b], y_hbm.at[row0 + r], ssem[b]).start()

        def wait_scatter(r):
            b = r % NBUF
            pltpu.make_async_copy(buf[b], y_hbm.at[0], ssem[b]).wait()

        for r in range(LEAD):
            issue_gather(r)
        for r in range(rpw):
            wait_gather(r)
            issue_scatter(r)
            nxt = r + LEAD
            if nxt < rpw:
                if nxt >= NBUF:
                    wait_scatter(nxt - NBUF)
                issue_gather(nxt)
        for r in range(rpw - NBUF, rpw):
            wait_scatter(r)

    return sc_gather


def _tc_body(y_ref, o_ref):
    o_ref[...] = y_ref[:, :50, :] * SCALE


def _tc_relayout(y, out_prev, k, rows, S, D):
    def body(y_ref, prev_ref, o_ref):
        _tc_body(y_ref, o_ref)

    grid = (rows // BLK,)
    y_spec = pl.BlockSpec((BLK, SP, D), lambda i: (i, 0, 0))
    prev_spec = pl.BlockSpec(memory_space=pl.ANY)
    o_spec = pl.BlockSpec((BLK, S, D), lambda i: (k * rows // BLK + i, 0, 0))
    return pl.pallas_call(
        body,
        grid=grid,
        in_specs=[y_spec, prev_spec],
        out_specs=o_spec,
        out_shape=jax.ShapeDtypeStruct((N_SPLIT * rows, S, D), jnp.float32),
        input_output_aliases={1: 0},
    )(y, out_prev)


def _tc_relayout_first(y, rows, S, D):
    grid = (rows // BLK,)
    y_spec = pl.BlockSpec((BLK, SP, D), lambda i: (i, 0, 0))
    o_spec = pl.BlockSpec((BLK, S, D), lambda i: (i, 0, 0))
    return pl.pallas_call(
        _tc_body,
        grid=grid,
        in_specs=[y_spec],
        out_specs=o_spec,
        out_shape=jax.ShapeDtypeStruct((N_SPLIT * rows, S, D), jnp.float32),
    )(y)


def kernel(x, target_vec, table, W, b):
    B, S = x.shape
    V, D = table.shape
    bc = B // N_SPLIT
    idx = x.reshape(N_SPLIT, _NW, bc // _NW, S).astype(jnp.int32)
    sc_gather = _make_sc_gather(bc, S, D)
    out = jnp.zeros((B, S, D), jnp.float32)
    for k in range(N_SPLIT):
        y = sc_gather(table, idx[k])
        upd = y[:, :S, :] * SCALE
        out = lax.dynamic_update_slice(out, upd, (k * bc, 0, 0))
        (out,) = lax.optimization_barrier((out,))
    return out


# speedup vs baseline: 1.9215x; 1.9215x over previous
import functools
import math

import jax
import jax.numpy as jnp
from jax import lax
from jax.experimental import pallas as pl
from jax.experimental.pallas import tpu as pltpu
from jax.experimental.pallas import tpu_sc as plsc

D_MODEL = 128
SCALE = math.sqrt(float(D_MODEL))

_NC = 2
_NS = 16
_NW = _NC * _NS
_L = 16

NBUF = 4


def _make_gather(B, S, D):
    assert B % _NW == 0
    rows_per_w = B // _NW
    n_groups = rows_per_w // NBUF
    assert rows_per_w % NBUF == 0 and n_groups >= 2

    mesh = plsc.VectorSubcoreMesh(core_axis_name="c", subcore_axis_name="s")

    @functools.partial(
        pl.kernel,
        mesh=mesh,
        out_type=jax.ShapeDtypeStruct((B, S, D), jnp.float32),
        scratch_types=[
            pltpu.VMEM((rows_per_w, S), jnp.int32),
            *([pltpu.VMEM((S, D), jnp.float32)] * NBUF),
            *([pltpu.VMEM((S, D), jnp.float32)] * NBUF),
            *([pltpu.SemaphoreType.DMA] * NBUF),
            *([pltpu.SemaphoreType.DMA] * NBUF),
        ],
    )
    def gather_kernel(table_hbm, idx_hbm, out_hbm, idx_v, *bufs):
        gbuf = bufs[:NBUF]
        obuf = bufs[NBUF:2 * NBUF]
        gsem = bufs[2 * NBUF:3 * NBUF]
        ssem = bufs[3 * NBUF:4 * NBUF]

        wid = lax.axis_index("s") * _NC + lax.axis_index("c")
        base = wid * rows_per_w
        pltpu.sync_copy(idx_hbm.at[wid], idx_v)

        def issue_gather(r, b):
            pltpu.make_async_copy(
                table_hbm.at[idx_v.at[r]], gbuf[b], gsem[b]).start()

        def issue_scatter(r, b):
            pltpu.make_async_copy(
                obuf[b], out_hbm.at[base + r], ssem[b]).start()

        def wait_gather(b):
            pltpu.make_async_copy(
                table_hbm.at[idx_v.at[0]], gbuf[b], gsem[b]).wait()

        def wait_scatter(b):
            pltpu.make_async_copy(
                obuf[b], out_hbm.at[base], ssem[b]).wait()

        def multiply(b):
            def row_body(r, carry):
                for j in range(D // _L):
                    sl = pl.ds(j * _L, _L)
                    obuf[b][r, sl] = gbuf[b][r, sl] * SCALE
                return carry

            lax.fori_loop(0, S, row_body, 0)

        for b in range(NBUF):
            issue_gather(b, b)
        for b in range(NBUF):
            wait_gather(b)
            multiply(b)
            issue_gather(NBUF + b, b)
            issue_scatter(b, b)

        def group_body(g, carry):
            for b in range(NBUF):
                r = g * NBUF + b
                wait_gather(b)
                wait_scatter(b)
                multiply(b)
                issue_gather(r + NBUF, b)
                issue_scatter(r, b)
            return carry

        lax.fori_loop(1, n_groups - 1, group_body, 0)

        for b in range(NBUF):
            r = (n_groups - 1) * NBUF + b
            wait_gather(b)
            wait_scatter(b)
            multiply(b)
            issue_scatter(r, b)

        for b in range(NBUF):
            wait_scatter(b)

    return gather_kernel


def kernel(x, target_vec, table, W, b):
    B, S = x.shape
    V, D = table.shape
    rows_per_w = B // _NW
    idx = x.reshape(_NW, rows_per_w, S).astype(jnp.int32)
    return _make_gather(B, S, D)(table, idx)
